# Initial kernel scaffold; baseline (speedup 1.0000x reference)
import jax
import jax.numpy as jnp
from jax.experimental import pallas as pl


def kernel(x, edge_index, edge_attr, weight, att_weight, bias):
    raise NotImplementedError("write your pallas kernel here")


# speedup vs baseline: 7.9965x; 7.9965x over previous
import functools

import jax
import jax.numpy as jnp
from jax import lax
from jax.experimental import pallas as pl
from jax.experimental.pallas import tpu as pltpu
from jax.experimental.pallas import tpu_sc as plsc

N = 10000
E = 320000
D = 128
C = 128

NC = 2
NS = 16
NW = NC * NS
EP = E // NW
CH = 128
NCH = EP // CH
MAIN = NCH * CH
TAIL = EP - MAIN
GRP = CH // 16
NPAD = 10240
RP = NPAD // NS

ROW_BLK = 1000
SLOPE = 0.2


def _proj_body(x_ref, w_ref, a2_ref, xw_ref, s12_ref):
    xw = jnp.dot(x_ref[...], w_ref[...], preferred_element_type=jnp.float32)
    xw_ref[...] = xw
    s12_ref[...] = jnp.dot(xw, a2_ref[...], preferred_element_type=jnp.float32)


def _project(x, weight, a2):
    return pl.pallas_call(
        _proj_body,
        grid=(N // ROW_BLK,),
        in_specs=[
            pl.BlockSpec((ROW_BLK, D), lambda i: (i, 0)),
            pl.BlockSpec((D, C), lambda i: (0, 0)),
            pl.BlockSpec((C, 2), lambda i: (0, 0)),
        ],
        out_specs=[
            pl.BlockSpec((ROW_BLK, C), lambda i: (i, 0)),
            pl.BlockSpec((ROW_BLK, 2), lambda i: (i, 0)),
        ],
        out_shape=[
            jax.ShapeDtypeStruct((N, C), jnp.float32),
            jax.ShapeDtypeStruct((N, 2), jnp.float32),
        ],
    )(x, weight, a2)


def _sc_body(xw_hbm, s1_hbm, s2_hbm, pack_hbm, z_hbm,
             part_hbm, alpha_hbm,
             acc, s1_v, s2_v, pack_v, alpha_db, rows_v, asc, sem):
    cid = lax.axis_index("c")
    sid = lax.axis_index("s")
    wid = cid * NS + sid

    pltpu.sync_copy(s1_hbm, s1_v)
    pltpu.sync_copy(s2_hbm, s2_v)
    pltpu.sync_copy(z_hbm, acc.at[pl.ds(sid * RP, RP)])
    plsc.subcore_barrier()

    def compute_group(g, nothing):
        rix = pack_v[0, pl.ds(g * 16, 16)]
        cix = pack_v[1, pl.ds(g * 16, 16)]
        eav = plsc.bitcast(pack_v[2, pl.ds(g * 16, 16)], jnp.float32)
        t = plsc.load_gather(s1_v, [rix]) + plsc.load_gather(s2_v, [cix])
        al = jnp.maximum(t, t * SLOPE) * eav
        alpha_db[0, pl.ds(g * 16, 16)] = al
        asc[pl.ds(0, 16)] = al
        asc[pl.ds(16, 16)] = al
        for e in range(16):
            sp = plsc.load_gather(asc, [jnp.full((16,), 16 + e, jnp.int32)])
            for f in range(C // 16):
                sl = pl.ds(f * 16, 16)
                rows_v[g * 16 + e, sl] = rows_v[g * 16 + e, sl] * sp
        return nothing

    def chunk(c, carry):
        pltpu.sync_copy(pack_hbm.at[wid, c], pack_v.at[pl.ds(0, 3)])
        pltpu.async_copy(xw_hbm.at[pack_v.at[1]], rows_v, sem).wait()
        lax.fori_loop(0, GRP, compute_group, 0)
        pltpu.sync_copy(alpha_db.at[0],
                        alpha_hbm.at[pl.ds(wid * EP + c * CH, CH)])
        pltpu.sync_copy(rows_v, acc.at[pack_v.at[0]], add=True)
        return carry

    lax.fori_loop(0, NCH, chunk, 0)

    pltpu.sync_copy(pack_hbm.at[wid, NCH], pack_v.at[pl.ds(0, 3)])
    rix = pack_v[0, pl.ds(0, 16)]
    cix = pack_v[1, pl.ds(0, 16)]
    eav = plsc.bitcast(pack_v[2, pl.ds(0, 16)], jnp.float32)
    pltpu.async_copy(xw_hbm.at[cix], rows_v.at[pl.ds(0, TAIL)], sem).wait()
    t = plsc.load_gather(s1_v, [rix]) + plsc.load_gather(s2_v, [cix])
    al = jnp.maximum(t, t * SLOPE) * eav
    asc[pl.ds(0, 16)] = al
    asc[pl.ds(16, 16)] = al
    pltpu.sync_copy(asc.at[pl.ds(0, 16)], alpha_hbm.at[pl.ds(wid * EP + MAIN, TAIL)])
    for e in range(16):
        sp = plsc.load_gather(asc, [jnp.full((16,), 16 + e, jnp.int32)])
        for f in range(C // 16):
            sl = pl.ds(f * 16, 16)
            rows_v[e, sl] = rows_v[e, sl] * sp
    pltpu.sync_copy(rows_v.at[pl.ds(0, TAIL)], acc.at[rix], add=True)

    plsc.subcore_barrier()

    pltpu.sync_copy(acc.at[pl.ds(sid * RP, RP)],
                    part_hbm.at[pl.ds(cid * NPAD + sid * RP, RP)])


def _sc_edges(xw, s1, s2, pack, zrows):
    mesh = plsc.VectorSubcoreMesh(core_axis_name="c", subcore_axis_name="s",
                                  num_cores=NC, num_subcores=NS)
    f = pl.kernel(
        _sc_body,
        out_type=[
            jax.ShapeDtypeStruct((NC * NPAD, C), jnp.float32),
            jax.ShapeDtypeStruct((E,), jnp.float32),
        ],
        mesh=mesh,
        scratch_types=[
            pltpu.VMEM_SHARED((NPAD, C), jnp.float32),
            pltpu.VMEM((N,), jnp.float32),
            pltpu.VMEM((N,), jnp.float32),
            pltpu.VMEM((6, CH), jnp.int32),
            pltpu.VMEM((2, CH), jnp.float32),
            pltpu.VMEM((CH, C), jnp.float32),
            pltpu.VMEM((32,), jnp.float32),
            pltpu.SemaphoreType.DMA,
        ],
        compiler_params=pltpu.CompilerParams(needs_layout_passes=False),
    )
    return f(xw, s1, s2, pack, zrows)


def _final_body(p0_ref, p1_ref, xw_ref, s12_ref, b_ref, out_ref, al_ref):
    t = s12_ref[:, 0:1] + s12_ref[:, 1:2]
    al = jnp.maximum(t, t * SLOPE)
    al_ref[...] = al
    out_ref[...] = (p0_ref[...] + p1_ref[...] + al * xw_ref[...] + b_ref[...])


def _finalize(p0, p1, xw, s12, bias2d):
    return pl.pallas_call(
        _final_body,
        grid=(N // ROW_BLK,),
        in_specs=[
            pl.BlockSpec((ROW_BLK, C), lambda i: (i, 0)),
            pl.BlockSpec((ROW_BLK, C), lambda i: (i, 0)),
            pl.BlockSpec((ROW_BLK, C), lambda i: (i, 0)),
            pl.BlockSpec((ROW_BLK, 2), lambda i: (i, 0)),
            pl.BlockSpec((1, C), lambda i: (0, 0)),
        ],
        out_specs=[
            pl.BlockSpec((ROW_BLK, C), lambda i: (i, 0)),
            pl.BlockSpec((ROW_BLK, 1), lambda i: (i, 0)),
        ],
        out_shape=[
            jax.ShapeDtypeStruct((N, C), jnp.float32),
            jax.ShapeDtypeStruct((N, 1), jnp.float32),
        ],
    )(p0, p1, xw, s12, bias2d)


def kernel(x, edge_index, edge_attr, weight, att_weight, bias):
    aw = att_weight.reshape(2 * C)
    a2 = jnp.stack([aw[:C], aw[C:]], axis=1)

    xw, s12 = _project(x, weight, a2)
    s1 = s12[:, 0]
    s2 = s12[:, 1]

    row32 = edge_index[0].astype(jnp.int32).reshape(NW, EP)
    col32 = edge_index[1].astype(jnp.int32).reshape(NW, EP)
    eabits = lax.bitcast_convert_type(
        edge_attr.reshape(-1).astype(jnp.float32), jnp.int32
    ).reshape(NW, EP)

    def _chunked(a):
        main = a[:, :MAIN].reshape(NW, NCH, CH)
        tail = jnp.pad(a[:, MAIN:], ((0, 0), (0, CH - TAIL)))[:, None, :]
        return jnp.concatenate([main, tail], axis=1)

    pack = jnp.stack(
        [_chunked(row32), _chunked(col32), _chunked(eabits)], axis=2
    )
    zrows = jnp.zeros((RP, C), jnp.float32)

    part, alpha_e = _sc_edges(xw, s1, s2, pack, zrows)

    out, alpha_loop = _finalize(part[:N], part[NPAD:NPAD + N], xw, s12,
                                bias.reshape(1, C))

    loop = jnp.arange(N, dtype=edge_index.dtype)
    edge_index_out = jnp.concatenate(
        [edge_index, jnp.stack([loop, loop])], axis=1)
    alpha = jnp.concatenate([alpha_e[:, None], alpha_loop], axis=0)
    return out, edge_index_out, alpha
